# baseline (speedup 1.0000x reference)
import jax
import jax.numpy as jnp
from jax import lax
from jax.experimental import pallas as pl
from jax.experimental.pallas import tpu as pltpu
from jax.experimental.pallas import tpu_sc as plsc

TOPK = 256
ROWS = 64
COLS = 8192
LANES = 16
NBINS = 2048
SHIFT = 32 - 11
SUFMASK = (1 << SHIFT) - 1
NCHUNK = COLS // LANES
NBCHUNK = NBINS // LANES
NWORKERS = 32
ROWS_PER_W = ROWS // NWORKERS


def _ikey(fvec):
    i = lax.bitcast_convert_type(fvec, jnp.int32)
    m = lax.shift_right_arithmetic(i, 31)
    flip = jnp.bitwise_and(jnp.bitwise_not(m), jnp.int32(0x7FFFFFFF))
    return jnp.bitwise_xor(i, flip)


def _scalar(vec):
    return lax.reduce_max(vec, axes=(0,))


def _lane0(vec):
    return lax.squeeze(lax.slice(vec, (0,), (1,)), (0,))


_GDN = lax.GatherDimensionNumbers(
    offset_dims=(), collapsed_slice_dims=(0,), start_index_map=(0,)
)


def _vgather(x, idx):
    return lax.gather(
        x,
        idx[:, None],
        dimension_numbers=_GDN,
        slice_sizes=(1,),
        mode=lax.GatherScatterMode.PROMISE_IN_BOUNDS,
    )


def _prefix16(x, iota16):
    for k in (1, 2, 4, 8):
        sh = _vgather(x, jnp.maximum(iota16 - k, 0))
        x = x + jnp.where(iota16 >= k, sh, 0)
    return x


def _topk_body(
    score_hbm, out_hbm, rows_v, outs_v, bins0_v, bins1_v, ck0_v, ck1_v,
    cp0_v, cp1_v, sem
):
    wid = lax.axis_index("s") * 2 + lax.axis_index("c")
    iota16 = lax.iota(jnp.int32, 16)
    zeros16 = jnp.zeros((LANES,), jnp.int32)
    ones16 = jnp.ones((LANES,), jnp.int32)
    onesf = jnp.ones((LANES,), jnp.float32)
    bins = (bins0_v, bins1_v)
    cks = (ck0_v, ck1_v)
    cps = (cp0_v, cp1_v)

    pltpu.sync_copy(
        score_hbm.at[pl.ds(wid * (ROWS_PER_W * COLS), ROWS_PER_W * COLS)], rows_v
    )

    def zero_bins(i, carry):
        bins0_v[pl.ds(i * LANES, LANES)] = zeros16
        bins1_v[pl.ds(i * LANES, LANES)] = zeros16
        return carry

    lax.fori_loop(0, NBCHUNK, zero_bins, 0, unroll=8)

    for r in range(ROWS_PER_W):
        def hist(i, carry, r=r):
            f = rows_v[pl.ds(r * COLS + i * LANES, LANES)]
            d = lax.shift_right_logical(_ikey(f), SHIFT)
            plsc.addupdate_scatter(bins[r], [d], ones16)
            return carry

        lax.fori_loop(0, NCHUNK, hist, 0, unroll=8)

    def make_scan(bins_ref):
        def scan_cond(c):
            i, _prev, found, _b, _rem = c
            return jnp.logical_and(i < NBCHUNK, jnp.logical_not(found))

        def scan_body(c):
            i, prev, found, b, rem = c
            bv = bins_ref[pl.ds(i * LANES, LANES)]
            cs = plsc.cumsum(bv)
            s = cs + prev
            crossed = s >= TOPK
            anyc = jnp.any(crossed)
            j = plsc.all_reduce_ffs(crossed)
            cum_b4 = _scalar(jnp.where(iota16 == j, s - bv, 0))
            b_new = i * LANES + _lane0(j)
            total = lax.squeeze(lax.slice(cs, (15,), (16,)), (0,))
            return (
                i + 1,
                prev + total,
                anyc,
                jnp.where(anyc, b_new, b),
                jnp.where(anyc, TOPK - cum_b4, rem),
            )

        _, _, _, b, rem = lax.while_loop(
            scan_cond,
            scan_body,
            (jnp.int32(0), jnp.int32(0), False, jnp.int32(0), jnp.int32(0)),
        )
        return b, rem

    bkt0, rem0 = make_scan(bins0_v)
    bkt1, rem1 = make_scan(bins1_v)
    bkt = (bkt0, bkt1)
    rem = (rem0, rem1)

    ncands = []
    for r in range(ROWS_PER_W):
        def mask_compact(i, n, r=r):
            f = rows_v[pl.ds(r * COLS + i * LANES, LANES)]
            ik = _ikey(f)
            d = lax.shift_right_logical(ik, SHIFT)
            outs_v[pl.ds(r * COLS + i * LANES, LANES)] = jnp.where(
                d < bkt[r], 1.0, 0.0
            )
            eq = d == bkt[r]
            cnt = _lane0(plsc.all_reduce_population_count(eq))

            @pl.when(cnt > 0)
            def _():
                suf = jnp.bitwise_and(ik, jnp.int32(SUFMASK))
                pos = iota16 + i * LANES
                plsc.store_compressed(cks[r].at[pl.ds(n, LANES)], suf, mask=eq)
                plsc.store_compressed(cps[r].at[pl.ds(n, LANES)], pos, mask=eq)

            return n + cnt

        ncands.append(
            lax.fori_loop(0, NCHUNK, mask_compact, jnp.int32(0), unroll=4)
        )

    for r in range(ROWS_PER_W):
        ncand = ncands[r]
        ck_v = cks[r]
        cp_v = cps[r]

        ck_v[pl.ds(ncand, LANES)] = jnp.full((LANES,), SUFMASK, jnp.int32)
        nvreg = (ncand + LANES - 1) // LANES

        def count_lt(t):
            def cnt(v, acc):
                s = ck_v[pl.ds(v * LANES, LANES)]
                c = s < t
                return acc + _lane0(plsc.all_reduce_population_count(c))

            return lax.fori_loop(0, nvreg, cnt, jnp.int32(0))

        def bit_step(k, prefix):
            bit = SHIFT - 1 - k
            cand = jnp.bitwise_or(prefix, lax.shift_left(jnp.int32(1), bit))
            return jnp.where(count_lt(cand) >= rem[r], prefix, cand)

        thr = lax.fori_loop(0, SHIFT, bit_step, jnp.int32(0))
        ties_needed = rem[r] - count_lt(thr)

        def fin(v, base):
            s = ck_v[pl.ds(v * LANES, LANES)]
            p = cp_v[pl.ds(v * LANES, LANES)]
            valid = (iota16 + v * LANES) < ncand
            lt = jnp.logical_and(s < thr, valid)
            eq = jnp.logical_and(s == thr, valid)
            rank = _prefix16(eq.astype(jnp.int32), iota16) + base
            sel = jnp.logical_or(lt, jnp.logical_and(eq, rank <= ties_needed))
            plsc.store_scatter(outs_v, [p + r * COLS], onesf, mask=sel)
            return base + _lane0(plsc.all_reduce_population_count(eq))

        lax.fori_loop(0, nvreg, fin, jnp.int32(0))

    pltpu.sync_copy(
        outs_v, out_hbm.at[pl.ds(wid * (ROWS_PER_W * COLS), ROWS_PER_W * COLS)]
    )


@jax.jit
def kernel(score):
    mesh = plsc.VectorSubcoreMesh(
        core_axis_name="c", subcore_axis_name="s", num_cores=2, num_subcores=16
    )
    flat = pl.kernel(
        _topk_body,
        out_type=jax.ShapeDtypeStruct((ROWS * COLS,), jnp.float32),
        mesh=mesh,
        compiler_params=pltpu.CompilerParams(needs_layout_passes=False),
        scratch_types=[
            pltpu.VMEM((ROWS_PER_W * COLS,), jnp.float32),
            pltpu.VMEM((ROWS_PER_W * COLS,), jnp.float32),
            pltpu.VMEM((NBINS,), jnp.int32),
            pltpu.VMEM((NBINS,), jnp.int32),
            pltpu.VMEM((COLS + LANES,), jnp.int32),
            pltpu.VMEM((COLS + LANES,), jnp.int32),
            pltpu.VMEM((COLS + LANES,), jnp.int32),
            pltpu.VMEM((COLS + LANES,), jnp.int32),
            pltpu.SemaphoreType.DMA,
        ],
    )(score.reshape(ROWS * COLS))
    return flat.reshape(ROWS, COLS)


# speedup vs baseline: 1.2621x; 1.2621x over previous
import jax
import jax.numpy as jnp
from jax import lax
from jax.experimental import pallas as pl
from jax.experimental.pallas import tpu as pltpu
from jax.experimental.pallas import tpu_sc as plsc

TOPK = 256
ROWS = 64
COLS = 8192
LANES = 16
NBINS = 2048
SHIFT = 32 - 11
SUFMASK = (1 << SHIFT) - 1
NCHUNK = COLS // LANES
NBCHUNK = NBINS // LANES
NWORKERS = 32
ROWS_PER_W = ROWS // NWORKERS


def _ikey(fvec):
    i = lax.bitcast_convert_type(fvec, jnp.int32)
    m = lax.shift_right_arithmetic(i, 31)
    flip = jnp.bitwise_and(jnp.bitwise_not(m), jnp.int32(0x7FFFFFFF))
    return jnp.bitwise_xor(i, flip)


def _scalar(vec):
    return lax.reduce_max(vec, axes=(0,))


def _lane0(vec):
    return lax.squeeze(lax.slice(vec, (0,), (1,)), (0,))


def _topk_body(score_hbm, out_hbm, row_v, out_v, bins_v, ckey_v, cpos_v, sem):
    wid = lax.axis_index("s") * 2 + lax.axis_index("c")
    iota16 = lax.iota(jnp.int32, 16)
    zeros16 = jnp.zeros((LANES,), jnp.int32)
    ones16 = jnp.ones((LANES,), jnp.int32)
    onesf = jnp.ones((LANES,), jnp.float32)

    for r in range(ROWS_PER_W):
        row = wid * ROWS_PER_W + r
        pltpu.sync_copy(score_hbm.at[row], row_v)

        def zero_bins(i, carry):
            bins_v[pl.ds(i * LANES, LANES)] = zeros16
            return carry

        lax.fori_loop(0, NBCHUNK, zero_bins, 0, unroll=8)

        def hist(i, carry):
            f = row_v[pl.ds(i * LANES, LANES)]
            d = lax.shift_right_logical(_ikey(f), SHIFT)
            plsc.addupdate_scatter(bins_v, [d], ones16)
            return carry

        lax.fori_loop(0, NCHUNK, hist, 0, unroll=8)

        def scan_cond(c):
            i, _prev, found, _b, _rem = c
            return jnp.logical_and(i < NBCHUNK, jnp.logical_not(found))

        def scan_body(c):
            i, prev, found, b, rem = c
            bv = bins_v[pl.ds(i * LANES, LANES)]
            cs = plsc.cumsum(bv)
            s = cs + prev
            crossed = s >= TOPK
            anyc = jnp.any(crossed)
            j = plsc.all_reduce_ffs(crossed)
            excl = s - bv
            cum_b4 = _scalar(jnp.where(iota16 == j, excl, 0))
            b_new = i * LANES + _lane0(j)
            rem_new = TOPK - cum_b4
            total = lax.squeeze(lax.slice(cs, (15,), (16,)), (0,))
            return (
                i + 1,
                prev + total,
                anyc,
                jnp.where(anyc, b_new, b),
                jnp.where(anyc, rem_new, rem),
            )

        _, _, _, bkt, rem = lax.while_loop(
            scan_cond,
            scan_body,
            (jnp.int32(0), jnp.int32(0), False, jnp.int32(0), jnp.int32(0)),
        )

        def mask_compact(i, n):
            f = row_v[pl.ds(i * LANES, LANES)]
            ik = _ikey(f)
            d = lax.shift_right_logical(ik, SHIFT)
            out_v[pl.ds(i * LANES, LANES)] = jnp.where(d < bkt, 1.0, 0.0)
            eq = d == bkt
            suf = jnp.bitwise_and(ik, jnp.int32(SUFMASK))
            pos = iota16 + i * LANES
            plsc.store_compressed(ckey_v.at[pl.ds(n, LANES)], suf, mask=eq)
            plsc.store_compressed(cpos_v.at[pl.ds(n, LANES)], pos, mask=eq)
            return n + _lane0(plsc.all_reduce_population_count(eq))

        ncand = lax.fori_loop(0, NCHUNK, mask_compact, jnp.int32(0), unroll=4)

        ckey_v[pl.ds(ncand, LANES)] = jnp.full((LANES,), SUFMASK, jnp.int32)
        nvreg = (ncand + LANES - 1) // LANES

        def count_lt(t):
            def cnt(v, acc):
                s = ckey_v[pl.ds(v * LANES, LANES)]
                c = s < t
                return acc + _lane0(plsc.all_reduce_population_count(c))

            return lax.fori_loop(0, nvreg, cnt, jnp.int32(0))

        def bit_step(k, prefix):
            bit = SHIFT - 1 - k
            cand = jnp.bitwise_or(prefix, lax.shift_left(jnp.int32(1), bit))
            return jnp.where(count_lt(cand) >= rem, prefix, cand)

        thr = lax.fori_loop(0, SHIFT, bit_step, jnp.int32(0))
        ties_needed = rem - count_lt(thr)

        def fin(v, base):
            s = ckey_v[pl.ds(v * LANES, LANES)]
            p = cpos_v[pl.ds(v * LANES, LANES)]
            valid = (iota16 + v * LANES) < ncand
            lt = jnp.logical_and(s < thr, valid)
            eq = jnp.logical_and(s == thr, valid)
            rank = plsc.cumsum(eq.astype(jnp.int32)) + base
            sel = jnp.logical_or(lt, jnp.logical_and(eq, rank <= ties_needed))
            plsc.store_scatter(out_v, [p], onesf, mask=sel)
            return base + _lane0(plsc.all_reduce_population_count(eq))

        lax.fori_loop(0, nvreg, fin, jnp.int32(0))

        pltpu.sync_copy(out_v, out_hbm.at[row])


@jax.jit
def kernel(score):
    mesh = plsc.VectorSubcoreMesh(
        core_axis_name="c", subcore_axis_name="s", num_cores=2, num_subcores=16
    )
    return pl.kernel(
        _topk_body,
        out_type=jax.ShapeDtypeStruct((ROWS, COLS), jnp.float32),
        mesh=mesh,
        compiler_params=pltpu.CompilerParams(needs_layout_passes=False),
        scratch_types=[
            pltpu.VMEM((COLS,), jnp.float32),
            pltpu.VMEM((COLS,), jnp.float32),
            pltpu.VMEM((NBINS,), jnp.int32),
            pltpu.VMEM((COLS + LANES,), jnp.int32),
            pltpu.VMEM((COLS + LANES,), jnp.int32),
            pltpu.SemaphoreType.DMA,
        ],
    )(score)


# speedup vs baseline: 1.2632x; 1.0008x over previous
import jax
import jax.numpy as jnp
from jax import lax
from jax.experimental import pallas as pl
from jax.experimental.pallas import tpu as pltpu
from jax.experimental.pallas import tpu_sc as plsc

TOPK = 256
ROWS = 64
COLS = 8192
LANES = 16
NBINS = 2048
SHIFT = 32 - 11
SUFMASK = (1 << SHIFT) - 1
NCHUNK = COLS // LANES
NBCHUNK = NBINS // LANES
NWORKERS = 32
ROWS_PER_W = ROWS // NWORKERS


def _ikey(fvec):
    i = lax.bitcast_convert_type(fvec, jnp.int32)
    m = lax.shift_right_arithmetic(i, 31)
    flip = jnp.bitwise_and(jnp.bitwise_not(m), jnp.int32(0x7FFFFFFF))
    return jnp.bitwise_xor(i, flip)


def _scalar(vec):
    return lax.reduce_max(vec, axes=(0,))


def _lane0(vec):
    return lax.squeeze(lax.slice(vec, (0,), (1,)), (0,))


def _topk_body(score_hbm, out_hbm, row_v, out_v, bins_v, ckey_v, cpos_v, sem):
    wid = lax.axis_index("s") * 2 + lax.axis_index("c")
    iota16 = lax.iota(jnp.int32, 16)
    zeros16 = jnp.zeros((LANES,), jnp.int32)
    ones16 = jnp.ones((LANES,), jnp.int32)
    onesf = jnp.ones((LANES,), jnp.float32)

    for r in range(ROWS_PER_W):
        row = wid * ROWS_PER_W + r
        pltpu.sync_copy(score_hbm.at[row], row_v)

        def zero_bins(i, carry):
            bins_v[pl.ds(i * LANES, LANES)] = zeros16
            return carry

        lax.fori_loop(0, NBCHUNK, zero_bins, 0, unroll=8)

        def hist(i, carry):
            f = row_v[pl.ds(i * LANES, LANES)]
            d = lax.shift_right_logical(_ikey(f), SHIFT)
            plsc.addupdate_scatter(bins_v, [d], ones16)
            return carry

        lax.fori_loop(0, NCHUNK, hist, 0, unroll=8)

        def scan_cond(c):
            i, _prev, found, _b, _rem = c
            return jnp.logical_and(i < NBCHUNK, jnp.logical_not(found))

        def scan_body(c):
            i, prev, found, b, rem = c
            bv = bins_v[pl.ds(i * LANES, LANES)]
            cs = plsc.cumsum(bv)
            s = cs + prev
            crossed = s >= TOPK
            anyc = jnp.any(crossed)
            j = plsc.all_reduce_ffs(crossed)
            excl = s - bv
            cum_b4 = _scalar(jnp.where(iota16 == j, excl, 0))
            b_new = i * LANES + _lane0(j)
            rem_new = TOPK - cum_b4
            total = lax.squeeze(lax.slice(cs, (15,), (16,)), (0,))
            return (
                i + 1,
                prev + total,
                anyc,
                jnp.where(anyc, b_new, b),
                jnp.where(anyc, rem_new, rem),
            )

        _, _, _, bkt, rem = lax.while_loop(
            scan_cond,
            scan_body,
            (jnp.int32(0), jnp.int32(0), False, jnp.int32(0), jnp.int32(0)),
        )

        def mask_compact(i, n):
            f = row_v[pl.ds(i * LANES, LANES)]
            ik = _ikey(f)
            d = lax.shift_right_logical(ik, SHIFT)
            out_v[pl.ds(i * LANES, LANES)] = jnp.where(d < bkt, 1.0, 0.0)
            eq = d == bkt
            suf = jnp.bitwise_and(ik, jnp.int32(SUFMASK))
            pos = iota16 + i * LANES
            plsc.store_compressed(ckey_v.at[pl.ds(n, LANES)], suf, mask=eq)
            plsc.store_compressed(cpos_v.at[pl.ds(n, LANES)], pos, mask=eq)
            return n + _lane0(plsc.all_reduce_population_count(eq))

        ncand = lax.fori_loop(0, NCHUNK, mask_compact, jnp.int32(0), unroll=4)

        ckey_v[pl.ds(ncand, LANES)] = jnp.full((LANES,), SUFMASK, jnp.int32)
        nvreg = (ncand + LANES - 1) // LANES

        def count_lt(t):
            def cnt(v, acc):
                s = ckey_v[pl.ds(v * LANES, LANES)]
                c = s < t
                return acc + _lane0(plsc.all_reduce_population_count(c))

            return lax.fori_loop(0, nvreg, cnt, jnp.int32(0))

        def bit_step(k, prefix):
            bit = SHIFT - 1 - k
            cand = jnp.bitwise_or(prefix, lax.shift_left(jnp.int32(1), bit))
            return jnp.where(count_lt(cand) >= rem, prefix, cand)

        thr = lax.fori_loop(0, SHIFT, bit_step, jnp.int32(0))
        ties_needed = rem - count_lt(thr)

        def fin(v, base):
            s = ckey_v[pl.ds(v * LANES, LANES)]
            p = cpos_v[pl.ds(v * LANES, LANES)]
            valid = (iota16 + v * LANES) < ncand
            lt = jnp.logical_and(s < thr, valid)
            eq = jnp.logical_and(s == thr, valid)
            rank = plsc.cumsum(eq.astype(jnp.int32)) + base
            sel = jnp.logical_or(lt, jnp.logical_and(eq, rank <= ties_needed))
            plsc.store_scatter(out_v, [p], onesf, mask=sel)
            return base + _lane0(plsc.all_reduce_population_count(eq))

        lax.fori_loop(0, nvreg, fin, jnp.int32(0))

        pltpu.sync_copy(out_v, out_hbm.at[row])


@jax.jit
def kernel(score):
    mesh = plsc.VectorSubcoreMesh(
        core_axis_name="c", subcore_axis_name="s", num_cores=2, num_subcores=16
    )
    return pl.kernel(
        _topk_body,
        out_type=jax.ShapeDtypeStruct((ROWS, COLS), jnp.float32),
        mesh=mesh,
        compiler_params=pltpu.CompilerParams(
            needs_layout_passes=False,
            disable_bounds_checks=True,
            disable_semaphore_checks=True,
        ),
        scratch_types=[
            pltpu.VMEM((COLS,), jnp.float32),
            pltpu.VMEM((COLS,), jnp.float32),
            pltpu.VMEM((NBINS,), jnp.int32),
            pltpu.VMEM((COLS + LANES,), jnp.int32),
            pltpu.VMEM((COLS + LANES,), jnp.int32),
            pltpu.SemaphoreType.DMA,
        ],
    )(score)
